# Initial kernel scaffold; baseline (speedup 1.0000x reference)
import jax
import jax.numpy as jnp
from jax.experimental import pallas as pl


def kernel(prediction, target):
    raise NotImplementedError("write your pallas kernel here")


# speedup vs baseline: 34.9253x; 34.9253x over previous
import functools

import jax
import jax.numpy as jnp
from jax.experimental import pallas as pl

_SIZE = 32000
_SMOOTHING = 0.1
_CONF = 1.0 - _SMOOTHING
_N = 2048
_CBLK = 1280


def _loss_kernel(tgt_ref, pred_ref, out_ref):
    k = pl.program_id(0)
    eps = jnp.float32(_SMOOTHING / (_SIZE - 2))
    log_eps = jnp.log(eps)
    conf = jnp.float32(_CONF)

    t = tgt_ref[...]
    t_is_zero = t == 0
    any_z = jnp.any(t_is_zero)
    any_nz = jnp.any(jnp.logical_not(t_is_zero))

    rid = jax.lax.broadcasted_iota(jnp.int32, (_N, 1), 0)
    w = jnp.where((rid == 0) & any_nz, 0.0, 1.0) * jnp.where(
        (rid == 1) & any_z, 0.0, 1.0
    )

    block = pred_ref[...]
    col = jax.lax.broadcasted_iota(jnp.int32, (_N, _CBLK), 1) + k * _CBLK
    sel = jnp.where(col == t, block, 0.0)
    rowsum = jnp.sum(block, axis=1, keepdims=True)
    psel = jnp.sum(sel, axis=1, keepdims=True)

    beta = jnp.where(t_is_zero, 0.0, eps - conf)
    partial = jnp.sum(w * (beta * psel - eps * rowsum))

    @pl.when(k == 0)
    def _init():
        a_i = jnp.where(
            t_is_zero,
            jnp.float32(_SIZE - 1) * eps * log_eps,
            jnp.float32(_SIZE - 2) * eps * log_eps + conf * jnp.log(conf),
        )
        p0 = block[:, 0:1]
        out_ref[...] = jnp.sum(w * (a_i + eps * p0)).reshape(1, 1)

    out_ref[...] += partial.reshape(1, 1)


@functools.partial(jax.jit, static_argnames=("interpret",))
def kernel(prediction, target, interpret=False):
    n, size = prediction.shape
    tgt2d = target.astype(jnp.int32).reshape(n, 1)
    out = pl.pallas_call(
        _loss_kernel,
        grid=(size // _CBLK,),
        in_specs=[
            pl.BlockSpec((n, 1), lambda k: (0, 0)),
            pl.BlockSpec((n, _CBLK), lambda k: (0, k)),
        ],
        out_specs=pl.BlockSpec((1, 1), lambda k: (0, 0)),
        out_shape=jax.ShapeDtypeStruct((1, 1), jnp.float32),
        interpret=interpret,
    )(tgt2d, prediction)
    return out[0, 0]


# speedup vs baseline: 35.6175x; 1.0198x over previous
import functools

import jax
import jax.numpy as jnp
from jax.experimental import pallas as pl

_SIZE = 32000
_SMOOTHING = 0.1
_CONF = 1.0 - _SMOOTHING
_N = 2048
_CBLK = 3200


def _loss_kernel(tgt_ref, pred_ref, out_ref):
    k = pl.program_id(0)
    eps = jnp.float32(_SMOOTHING / (_SIZE - 2))
    log_eps = jnp.log(eps)
    conf = jnp.float32(_CONF)

    t = tgt_ref[...]
    t_is_zero = t == 0
    any_z = jnp.any(t_is_zero)
    any_nz = jnp.any(jnp.logical_not(t_is_zero))

    rid = jax.lax.broadcasted_iota(jnp.int32, (_N, 1), 0)
    w = jnp.where((rid == 0) & any_nz, 0.0, 1.0) * jnp.where(
        (rid == 1) & any_z, 0.0, 1.0
    )

    block = pred_ref[...]
    col = jax.lax.broadcasted_iota(jnp.int32, (_N, _CBLK), 1) + k * _CBLK
    sel = jnp.where(col == t, block, 0.0)
    rowsum = jnp.sum(block, axis=1, keepdims=True)
    psel = jnp.sum(sel, axis=1, keepdims=True)

    beta = jnp.where(t_is_zero, 0.0, eps - conf)
    partial = jnp.sum(w * (beta * psel - eps * rowsum))

    @pl.when(k == 0)
    def _init():
        a_i = jnp.where(
            t_is_zero,
            jnp.float32(_SIZE - 1) * eps * log_eps,
            jnp.float32(_SIZE - 2) * eps * log_eps + conf * jnp.log(conf),
        )
        p0 = block[:, 0:1]
        out_ref[...] = jnp.sum(w * (a_i + eps * p0)).reshape(1, 1)

    out_ref[...] += partial.reshape(1, 1)


@functools.partial(jax.jit, static_argnames=("interpret",))
def kernel(prediction, target, interpret=False):
    n, size = prediction.shape
    tgt2d = target.astype(jnp.int32).reshape(n, 1)
    out = pl.pallas_call(
        _loss_kernel,
        grid=(size // _CBLK,),
        in_specs=[
            pl.BlockSpec((n, 1), lambda k: (0, 0)),
            pl.BlockSpec((n, _CBLK), lambda k: (0, k)),
        ],
        out_specs=pl.BlockSpec((1, 1), lambda k: (0, 0)),
        out_shape=jax.ShapeDtypeStruct((1, 1), jnp.float32),
        interpret=interpret,
    )(tgt2d, prediction)
    return out[0, 0]
